# Initial kernel scaffold; baseline (speedup 1.0000x reference)
import jax
import jax.numpy as jnp
from jax.experimental import pallas as pl


def kernel(x, emb_table):
    raise NotImplementedError("write your pallas kernel here")


# speedup vs baseline: 1.4596x; 1.4596x over previous
import jax
import jax.numpy as jnp
from jax.experimental import pallas as pl


_BS = 256


def _add_kernel(x_ref, emb_ref, out_ref):
    out_ref[...] = x_ref[...] + emb_ref[...]


def kernel(x, emb_table):
    batch, seq, d = x.shape
    pos = emb_table[:seq]
    bs = _BS if seq % _BS == 0 else seq
    grid = (seq // bs, batch)
    return pl.pallas_call(
        _add_kernel,
        grid=grid,
        in_specs=[
            pl.BlockSpec((1, bs, d), lambda i, j: (j, i, 0)),
            pl.BlockSpec((bs, d), lambda i, j: (i, 0)),
        ],
        out_specs=pl.BlockSpec((1, bs, d), lambda i, j: (j, i, 0)),
        out_shape=jax.ShapeDtypeStruct((batch, seq, d), x.dtype),
    )(x, pos)


# speedup vs baseline: 1.9328x; 1.3242x over previous
import jax
import jax.numpy as jnp
from jax.experimental import pallas as pl


_BS = 512


def _add_kernel(x_ref, emb_ref, out_ref):
    out_ref[...] = x_ref[...] + emb_ref[...]


def kernel(x, emb_table):
    batch, seq, d = x.shape
    pos = emb_table[:seq]
    bs = _BS if seq % _BS == 0 else seq
    grid = (seq // bs, batch)
    return pl.pallas_call(
        _add_kernel,
        grid=grid,
        in_specs=[
            pl.BlockSpec((1, bs, d), lambda i, j: (j, i, 0)),
            pl.BlockSpec((bs, d), lambda i, j: (i, 0)),
        ],
        out_specs=pl.BlockSpec((1, bs, d), lambda i, j: (j, i, 0)),
        out_shape=jax.ShapeDtypeStruct((batch, seq, d), x.dtype),
    )(x, pos)


# speedup vs baseline: 2.1064x; 1.0899x over previous
import jax
import jax.numpy as jnp
from jax.experimental import pallas as pl


_BS = 1024


def _add_kernel(x_ref, emb_ref, out_ref):
    out_ref[...] = x_ref[...] + emb_ref[...]


def kernel(x, emb_table):
    batch, seq, d = x.shape
    pos = emb_table[:seq]
    bs = _BS if seq % _BS == 0 else seq
    grid = (seq // bs, batch)
    return pl.pallas_call(
        _add_kernel,
        grid=grid,
        in_specs=[
            pl.BlockSpec((1, bs, d), lambda i, j: (j, i, 0)),
            pl.BlockSpec((bs, d), lambda i, j: (i, 0)),
        ],
        out_specs=pl.BlockSpec((1, bs, d), lambda i, j: (j, i, 0)),
        out_shape=jax.ShapeDtypeStruct((batch, seq, d), x.dtype),
    )(x, pos)


# speedup vs baseline: 2.2796x; 1.0822x over previous
import jax
import jax.numpy as jnp
from jax.experimental import pallas as pl


_BS = 2048


def _add_kernel(x_ref, emb_ref, out_ref):
    out_ref[...] = x_ref[...] + emb_ref[...]


def kernel(x, emb_table):
    batch, seq, d = x.shape
    pos = emb_table[:seq]
    bs = _BS if seq % _BS == 0 else seq
    grid = (seq // bs, batch)
    return pl.pallas_call(
        _add_kernel,
        grid=grid,
        in_specs=[
            pl.BlockSpec((1, bs, d), lambda i, j: (j, i, 0)),
            pl.BlockSpec((bs, d), lambda i, j: (i, 0)),
        ],
        out_specs=pl.BlockSpec((1, bs, d), lambda i, j: (j, i, 0)),
        out_shape=jax.ShapeDtypeStruct((batch, seq, d), x.dtype),
    )(x, pos)
